# baseline (speedup 1.0000x reference)
import functools
import jax
import jax.numpy as jnp
from jax import lax
from jax.experimental import pallas as pl
from jax.experimental.pallas import tpu as pltpu
from jax.experimental.pallas import tpu_sc as plsc

B = 8
N = 2048

NC, NS, L = 2, 16, 16
NW = NC * NS
N_SC = 2
N_TC = B - N_SC
Q = NW // N_SC
COLS_W = N // Q
CB = 8
BLOCKS = COLS_W // (CB * L)

_mesh = plsc.VectorSubcoreMesh(core_axis_name="c", subcore_axis_name="s",
                               num_cores=NC, num_subcores=NS)


def _min_pass(src_planes, dst_planes, out_v, q):
    sx_v, sy_v, sz_v, sn_v = src_planes
    dx_v, dy_v, dz_v, dn_v = dst_planes
    for blk in range(BLOCKS):
        colbase = q * COLS_W + blk * (CB * L)
        gx = [dx_v[pl.ds(colbase + v * L, L)] * -2.0 for v in range(CB)]
        gy = [dy_v[pl.ds(colbase + v * L, L)] * -2.0 for v in range(CB)]
        gz = [dz_v[pl.ds(colbase + v * L, L)] * -2.0 for v in range(CB)]
        init = tuple(jnp.full((L,), jnp.inf, jnp.float32) for _ in range(CB))
        bidx = [jnp.full((L,), u, jnp.int32) for u in range(L)]

        @plsc.parallel_loop(0, N // L, unroll=1, carry=init)
        def acc(ci, acc):
            base = ci * L
            sxv = sx_v[pl.ds(base, L)]
            syv = sy_v[pl.ds(base, L)]
            szv = sz_v[pl.ds(base, L)]
            snv = sn_v[pl.ds(base, L)]
            for u in range(L):
                px = sxv.at[bidx[u]].get(mode="promise_in_bounds")
                py = syv.at[bidx[u]].get(mode="promise_in_bounds")
                pz = szv.at[bidx[u]].get(mode="promise_in_bounds")
                pn = snv.at[bidx[u]].get(mode="promise_in_bounds")
                acc = tuple(
                    jnp.minimum(acc[v],
                                px * gx[v] + py * gy[v] + pz * gz[v] + pn)
                    for v in range(CB))
            return acc
        for v in range(CB):
            gn = dn_v[pl.ds(colbase + v * L, L)]
            out_v[pl.ds(blk * CB * L + v * L, L)] = acc[v] + gn


@functools.partial(
    pl.kernel,
    out_type=[jax.ShapeDtypeStruct((N_SC, N), jnp.float32),
              jax.ShapeDtypeStruct((N_SC, N), jnp.float32)],
    mesh=_mesh,
    scratch_types=(
        [pltpu.VMEM((N,), jnp.float32) for _ in range(8)]
        + [pltpu.VMEM((COLS_W,), jnp.float32),
           pltpu.VMEM((COLS_W,), jnp.float32)]
    ),
    compiler_params=pltpu.CompilerParams(needs_layout_passes=False),
)
def _sc_minsq(p_hbm, g_hbm, z1_hbm, z2_hbm,
              px_v, py_v, pz_v, pn_v, gx_v, gy_v, gz_v, gn_v, o1_v, o2_v):
    wid = lax.axis_index("c") * NS + lax.axis_index("s")
    b = wid // Q
    q = wid % Q
    p_planes = (px_v, py_v, pz_v, pn_v)
    g_planes = (gx_v, gy_v, gz_v, gn_v)
    for c in range(4):
        pltpu.sync_copy(p_hbm.at[b, c], p_planes[c])
        pltpu.sync_copy(g_hbm.at[b, c], g_planes[c])
    _min_pass(p_planes, g_planes, o1_v, q)
    pltpu.sync_copy(o1_v, z1_hbm.at[b, pl.ds(q * COLS_W, COLS_W)])
    _min_pass(g_planes, p_planes, o2_v, q)
    pltpu.sync_copy(o2_v, z2_hbm.at[b, pl.ds(q * COLS_W, COLS_W)])


ROWS = 1024
T = N // ROWS


def _tc_body(pt_ref, g_ref, loss_ref, zmin_ref):
    b = pl.program_id(0)
    t = pl.program_id(1)
    p = jnp.transpose(pt_ref[0])
    g = g_ref[0]
    d = ((p[:, 0:1] - g[0:1, :]) ** 2
         + (p[:, 1:2] - g[1:2, :]) ** 2
         + (p[:, 2:3] - g[2:3, :]) ** 2)

    colmin = jnp.min(d, axis=0, keepdims=True)
    zmin_new = jnp.where(t == 0, colmin,
                         jnp.minimum(zmin_ref[...], colmin))
    zmin_ref[...] = zmin_new

    partial = jnp.sum(jnp.sqrt(jnp.min(d, axis=1)))

    last_t = t == T - 1
    inc = partial + jnp.where(last_t, jnp.sum(jnp.sqrt(zmin_new)), 0.0)
    first = jnp.logical_and(b == 0, t == 0)
    loss_ref[0, 0] = jnp.where(first, 0.0, loss_ref[0, 0]) + inc


def _tc_chamfer(predict_pc, gt_pc):
    return pl.pallas_call(
        _tc_body,
        grid=(N_TC, T),
        in_specs=[
            pl.BlockSpec((1, 3, ROWS), lambda b, t: (b + N_SC, 0, t)),
            pl.BlockSpec((1, 3, N), lambda b, t: (b + N_SC, 0, 0)),
        ],
        out_specs=pl.BlockSpec((1, 1), lambda b, t: (0, 0),
                               memory_space=pltpu.SMEM),
        out_shape=jax.ShapeDtypeStruct((1, 1), jnp.float32),
        scratch_shapes=[pltpu.VMEM((1, N), jnp.float32)],
    )(predict_pc, gt_pc)


def _finish_body(z1_ref, z2_ref, tc_ref, out_ref):
    s = (jnp.sum(jnp.sqrt(jnp.maximum(z1_ref[...], 0.0)))
         + jnp.sum(jnp.sqrt(jnp.maximum(z2_ref[...], 0.0))))
    out_ref[0, 0] = (s + tc_ref[0, 0]) * (1.0 / (B * N))


def _finish(z1, z2, tc_part):
    out = pl.pallas_call(
        _finish_body,
        in_specs=[pl.BlockSpec(memory_space=pltpu.VMEM),
                  pl.BlockSpec(memory_space=pltpu.VMEM),
                  pl.BlockSpec(memory_space=pltpu.SMEM)],
        out_specs=pl.BlockSpec(memory_space=pltpu.SMEM),
        out_shape=jax.ShapeDtypeStruct((1, 1), jnp.float32),
    )(z1, z2, tc_part)
    return out[0, 0]


def kernel(predict_pc, gt_pc):
    p_sc = predict_pc[:N_SC]
    g_sc = gt_pc[:N_SC]
    pp = jnp.concatenate(
        [p_sc, jnp.sum(p_sc * p_sc, axis=1, keepdims=True)], axis=1)
    gp = jnp.concatenate(
        [g_sc, jnp.sum(g_sc * g_sc, axis=1, keepdims=True)], axis=1)
    z1, z2 = _sc_minsq(pp, gp)
    tc_part = _tc_chamfer(predict_pc, gt_pc)
    return _finish(z1, z2, tc_part)


# speedup vs baseline: 5.7623x; 5.7623x over previous
import functools
import jax
import jax.numpy as jnp
from jax import lax
from jax.experimental import pallas as pl
from jax.experimental.pallas import tpu as pltpu
from jax.experimental.pallas import tpu_sc as plsc

B = 8
N = 2048

NC, NS, L = 2, 16, 16
NW = NC * NS
N_SC = 1
N_TC = B - N_SC
Q = NW // N_SC
COLS_W = N // Q
CB = 4
BLOCKS = COLS_W // (CB * L)

_mesh = plsc.VectorSubcoreMesh(core_axis_name="c", subcore_axis_name="s",
                               num_cores=NC, num_subcores=NS)


def _min_pass(src_planes, dst_planes, out_v, q):
    sx_v, sy_v, sz_v, sn_v = src_planes
    dx_v, dy_v, dz_v, dn_v = dst_planes
    for blk in range(BLOCKS):
        colbase = q * COLS_W + blk * (CB * L)
        gx = [dx_v[pl.ds(colbase + v * L, L)] * -2.0 for v in range(CB)]
        gy = [dy_v[pl.ds(colbase + v * L, L)] * -2.0 for v in range(CB)]
        gz = [dz_v[pl.ds(colbase + v * L, L)] * -2.0 for v in range(CB)]
        init = tuple(jnp.full((L,), jnp.inf, jnp.float32) for _ in range(CB))

        @plsc.parallel_loop(0, N, unroll=2, carry=init)
        def acc(i, acc):
            ridx = jnp.full((L,), i, jnp.int32)
            px = plsc.load_gather(sx_v, [ridx])
            py = plsc.load_gather(sy_v, [ridx])
            pz = plsc.load_gather(sz_v, [ridx])
            pn = plsc.load_gather(sn_v, [ridx])
            return tuple(
                jnp.minimum(acc[v],
                            px * gx[v] + py * gy[v] + pz * gz[v] + pn)
                for v in range(CB))
        for v in range(CB):
            gn = dn_v[pl.ds(colbase + v * L, L)]
            out_v[pl.ds(blk * CB * L + v * L, L)] = acc[v] + gn


@functools.partial(
    pl.kernel,
    out_type=[jax.ShapeDtypeStruct((N_SC, N), jnp.float32),
              jax.ShapeDtypeStruct((N_SC, N), jnp.float32)],
    mesh=_mesh,
    scratch_types=(
        [pltpu.VMEM((N,), jnp.float32) for _ in range(8)]
        + [pltpu.VMEM((COLS_W,), jnp.float32),
           pltpu.VMEM((COLS_W,), jnp.float32)]
    ),
    compiler_params=pltpu.CompilerParams(needs_layout_passes=False),
)
def _sc_minsq(p_hbm, g_hbm, z1_hbm, z2_hbm,
              px_v, py_v, pz_v, pn_v, gx_v, gy_v, gz_v, gn_v, o1_v, o2_v):
    wid = lax.axis_index("c") * NS + lax.axis_index("s")
    b = wid // Q
    q = wid % Q
    p_planes = (px_v, py_v, pz_v, pn_v)
    g_planes = (gx_v, gy_v, gz_v, gn_v)
    for c in range(4):
        pltpu.sync_copy(p_hbm.at[b, c], p_planes[c])
        pltpu.sync_copy(g_hbm.at[b, c], g_planes[c])
    _min_pass(p_planes, g_planes, o1_v, q)
    pltpu.sync_copy(o1_v, z1_hbm.at[b, pl.ds(q * COLS_W, COLS_W)])
    _min_pass(g_planes, p_planes, o2_v, q)
    pltpu.sync_copy(o2_v, z2_hbm.at[b, pl.ds(q * COLS_W, COLS_W)])


ROWS = 1024
T = N // ROWS


def _tc_body(pt_ref, g_ref, loss_ref, zmin_ref):
    b = pl.program_id(0)
    t = pl.program_id(1)
    p = jnp.transpose(pt_ref[0])
    g = g_ref[0]
    d = ((p[:, 0:1] - g[0:1, :]) ** 2
         + (p[:, 1:2] - g[1:2, :]) ** 2
         + (p[:, 2:3] - g[2:3, :]) ** 2)

    colmin = jnp.min(d, axis=0, keepdims=True)
    zmin_new = jnp.where(t == 0, colmin,
                         jnp.minimum(zmin_ref[...], colmin))
    zmin_ref[...] = zmin_new

    partial = jnp.sum(jnp.sqrt(jnp.min(d, axis=1)))

    last_t = t == T - 1
    inc = partial + jnp.where(last_t, jnp.sum(jnp.sqrt(zmin_new)), 0.0)
    first = jnp.logical_and(b == 0, t == 0)
    loss_ref[0, 0] = jnp.where(first, 0.0, loss_ref[0, 0]) + inc


def _tc_chamfer(predict_pc, gt_pc):
    return pl.pallas_call(
        _tc_body,
        grid=(N_TC, T),
        in_specs=[
            pl.BlockSpec((1, 3, ROWS), lambda b, t: (b + N_SC, 0, t)),
            pl.BlockSpec((1, 3, N), lambda b, t: (b + N_SC, 0, 0)),
        ],
        out_specs=pl.BlockSpec((1, 1), lambda b, t: (0, 0),
                               memory_space=pltpu.SMEM),
        out_shape=jax.ShapeDtypeStruct((1, 1), jnp.float32),
        scratch_shapes=[pltpu.VMEM((1, N), jnp.float32)],
    )(predict_pc, gt_pc)


def _finish_body(z1_ref, z2_ref, tc_ref, out_ref):
    s = (jnp.sum(jnp.sqrt(jnp.maximum(z1_ref[...], 0.0)))
         + jnp.sum(jnp.sqrt(jnp.maximum(z2_ref[...], 0.0))))
    out_ref[0, 0] = (s + tc_ref[0, 0]) * (1.0 / (B * N))


def _finish(z1, z2, tc_part):
    out = pl.pallas_call(
        _finish_body,
        in_specs=[pl.BlockSpec(memory_space=pltpu.VMEM),
                  pl.BlockSpec(memory_space=pltpu.VMEM),
                  pl.BlockSpec(memory_space=pltpu.SMEM)],
        out_specs=pl.BlockSpec(memory_space=pltpu.SMEM),
        out_shape=jax.ShapeDtypeStruct((1, 1), jnp.float32),
    )(z1, z2, tc_part)
    return out[0, 0]


def kernel(predict_pc, gt_pc):
    p_sc = predict_pc[:N_SC]
    g_sc = gt_pc[:N_SC]
    pp = jnp.concatenate(
        [p_sc, jnp.sum(p_sc * p_sc, axis=1, keepdims=True)], axis=1)
    gp = jnp.concatenate(
        [g_sc, jnp.sum(g_sc * g_sc, axis=1, keepdims=True)], axis=1)
    z1, z2 = _sc_minsq(pp, gp)
    tc_part = _tc_chamfer(predict_pc, gt_pc)
    return _finish(z1, z2, tc_part)


# speedup vs baseline: 5.9381x; 1.0305x over previous
import functools
import jax
import jax.numpy as jnp
from jax import lax
from jax.experimental import pallas as pl
from jax.experimental.pallas import tpu as pltpu
from jax.experimental.pallas import tpu_sc as plsc

B = 8
N = 2048

NC, NS, L = 2, 16, 16
NW = NC * NS
N_SC = 1
N_TC = B - N_SC
Q = NW // N_SC
COLS_W = N // Q
CB = 4
BLOCKS = COLS_W // (CB * L)

_mesh = plsc.VectorSubcoreMesh(core_axis_name="c", subcore_axis_name="s",
                               num_cores=NC, num_subcores=NS)


def _min_pass(src_planes, dst_planes, out_v, q):
    sx_v, sy_v, sz_v, sn_v = src_planes
    dx_v, dy_v, dz_v, dn_v = dst_planes
    for blk in range(BLOCKS):
        colbase = q * COLS_W + blk * (CB * L)
        gx = [dx_v[pl.ds(colbase + v * L, L)] * -2.0 for v in range(CB)]
        gy = [dy_v[pl.ds(colbase + v * L, L)] * -2.0 for v in range(CB)]
        gz = [dz_v[pl.ds(colbase + v * L, L)] * -2.0 for v in range(CB)]
        init = tuple(jnp.full((L,), jnp.inf, jnp.float32) for _ in range(CB))

        @plsc.parallel_loop(0, N, unroll=2, carry=init)
        def acc(i, acc):
            ridx = jnp.full((L,), i, jnp.int32)
            px = plsc.load_gather(sx_v, [ridx])
            py = plsc.load_gather(sy_v, [ridx])
            pz = plsc.load_gather(sz_v, [ridx])
            pn = plsc.load_gather(sn_v, [ridx])
            return tuple(
                jnp.minimum(acc[v],
                            px * gx[v] + py * gy[v] + pz * gz[v] + pn)
                for v in range(CB))
        for v in range(CB):
            gn = dn_v[pl.ds(colbase + v * L, L)]
            out_v[pl.ds(blk * CB * L + v * L, L)] = acc[v] + gn


@functools.partial(
    pl.kernel,
    out_type=[jax.ShapeDtypeStruct((N_SC, N), jnp.float32),
              jax.ShapeDtypeStruct((N_SC, N), jnp.float32)],
    mesh=_mesh,
    scratch_types=(
        [pltpu.VMEM((N,), jnp.float32) for _ in range(8)]
        + [pltpu.VMEM((COLS_W,), jnp.float32),
           pltpu.VMEM((COLS_W,), jnp.float32)]
    ),
    compiler_params=pltpu.CompilerParams(needs_layout_passes=False),
)
def _sc_minsq(p_hbm, g_hbm, z1_hbm, z2_hbm,
              px_v, py_v, pz_v, pn_v, gx_v, gy_v, gz_v, gn_v, o1_v, o2_v):
    wid = lax.axis_index("c") * NS + lax.axis_index("s")
    b = wid // Q
    q = wid % Q
    p_planes = (px_v, py_v, pz_v, pn_v)
    g_planes = (gx_v, gy_v, gz_v, gn_v)
    for c in range(4):
        pltpu.sync_copy(p_hbm.at[b, c], p_planes[c])
        pltpu.sync_copy(g_hbm.at[b, c], g_planes[c])
    _min_pass(p_planes, g_planes, o1_v, q)
    pltpu.sync_copy(o1_v, z1_hbm.at[b, pl.ds(q * COLS_W, COLS_W)])
    _min_pass(g_planes, p_planes, o2_v, q)
    pltpu.sync_copy(o2_v, z2_hbm.at[b, pl.ds(q * COLS_W, COLS_W)])


ROWS = 2048
T = N // ROWS


def _tc_body(pt_ref, g_ref, loss_ref, zmin_ref):
    b = pl.program_id(0)
    t = pl.program_id(1)
    p = jnp.transpose(pt_ref[0])
    g = g_ref[0]
    d = ((p[:, 0:1] - g[0:1, :]) ** 2
         + (p[:, 1:2] - g[1:2, :]) ** 2
         + (p[:, 2:3] - g[2:3, :]) ** 2)

    colmin = jnp.min(d, axis=0, keepdims=True)
    zmin_new = jnp.where(t == 0, colmin,
                         jnp.minimum(zmin_ref[...], colmin))
    zmin_ref[...] = zmin_new

    partial = jnp.sum(jnp.sqrt(jnp.min(d, axis=1)))

    last_t = t == T - 1
    inc = partial + jnp.where(last_t, jnp.sum(jnp.sqrt(zmin_new)), 0.0)
    first = jnp.logical_and(b == 0, t == 0)
    loss_ref[0, 0] = jnp.where(first, 0.0, loss_ref[0, 0]) + inc


def _tc_chamfer(predict_pc, gt_pc):
    return pl.pallas_call(
        _tc_body,
        grid=(N_TC, T),
        in_specs=[
            pl.BlockSpec((1, 3, ROWS), lambda b, t: (b + N_SC, 0, t)),
            pl.BlockSpec((1, 3, N), lambda b, t: (b + N_SC, 0, 0)),
        ],
        out_specs=pl.BlockSpec((1, 1), lambda b, t: (0, 0),
                               memory_space=pltpu.SMEM),
        out_shape=jax.ShapeDtypeStruct((1, 1), jnp.float32),
        scratch_shapes=[pltpu.VMEM((1, N), jnp.float32)],
    )(predict_pc, gt_pc)


def _finish_body(z1_ref, z2_ref, tc_ref, out_ref):
    s = (jnp.sum(jnp.sqrt(jnp.maximum(z1_ref[...], 0.0)))
         + jnp.sum(jnp.sqrt(jnp.maximum(z2_ref[...], 0.0))))
    out_ref[0, 0] = (s + tc_ref[0, 0]) * (1.0 / (B * N))


def _finish(z1, z2, tc_part):
    out = pl.pallas_call(
        _finish_body,
        in_specs=[pl.BlockSpec(memory_space=pltpu.VMEM),
                  pl.BlockSpec(memory_space=pltpu.VMEM),
                  pl.BlockSpec(memory_space=pltpu.SMEM)],
        out_specs=pl.BlockSpec(memory_space=pltpu.SMEM),
        out_shape=jax.ShapeDtypeStruct((1, 1), jnp.float32),
    )(z1, z2, tc_part)
    return out[0, 0]


def kernel(predict_pc, gt_pc):
    p_sc = predict_pc[:N_SC]
    g_sc = gt_pc[:N_SC]
    pp = jnp.concatenate(
        [p_sc, jnp.sum(p_sc * p_sc, axis=1, keepdims=True)], axis=1)
    gp = jnp.concatenate(
        [g_sc, jnp.sum(g_sc * g_sc, axis=1, keepdims=True)], axis=1)
    z1, z2 = _sc_minsq(pp, gp)
    tc_part = _tc_chamfer(predict_pc, gt_pc)
    return _finish(z1, z2, tc_part)
